# baseline (speedup 1.0000x reference)
import functools

import jax
import jax.numpy as jnp
from jax.experimental import pallas as pl
from jax.experimental.pallas import tpu as pltpu

NUM_OPS = 8

BT = 1024
BF = 512


def _route_body(x_ref, o_ref):
    lane = jax.lax.broadcasted_iota(jnp.int32, (1, 128), 1)
    masked = jnp.where(lane < NUM_OPS, x_ref[...], -jnp.inf)
    m = jnp.max(masked)
    idx = jnp.min(jnp.where(masked == m, lane, 128))
    o_ref[0, 0] = idx


def _route(x):
    xs = jax.lax.slice(x, (0, 0, 0), (1, 1, 128)).reshape(1, 128)
    out = pl.pallas_call(
        _route_body,
        out_shape=jax.ShapeDtypeStruct((1, 1), jnp.int32),
        out_specs=pl.BlockSpec(memory_space=pltpu.SMEM),
    )(xs)
    return out.reshape((1,))


def _ffn_body(e_ref, x_ref, w1_ref, b1_ref, w2_ref, b2_ref, o_ref):
    f = pl.program_id(1)
    h = jnp.maximum(
        jnp.dot(x_ref[...], w1_ref[0], preferred_element_type=jnp.float32)
        + b1_ref[0], 0.0)
    p = jnp.dot(h, w2_ref[0], preferred_element_type=jnp.float32)

    @pl.when(f == 0)
    def _():
        o_ref[...] = p + b2_ref[0]

    @pl.when(f > 0)
    def _():
        o_ref[...] += p


@jax.jit
def kernel(x, W1, b1, W2, b2):
    B, S, D = x.shape
    E, _, F = W1.shape
    tokens = B * S
    x2 = x.reshape(tokens, D)
    b1r = b1.reshape(E, 1, F)
    b2r = b2.reshape(E, 1, D)
    e_idx = _route(x)

    grid = (tokens // BT, F // BF)
    grid_spec = pltpu.PrefetchScalarGridSpec(
        num_scalar_prefetch=1,
        grid=grid,
        in_specs=[
            pl.BlockSpec((BT, D), lambda t, f, e: (t, 0)),
            pl.BlockSpec((1, D, BF), lambda t, f, e: (e[0], 0, f)),
            pl.BlockSpec((1, 1, BF), lambda t, f, e: (e[0], 0, f)),
            pl.BlockSpec((1, BF, D), lambda t, f, e: (e[0], f, 0)),
            pl.BlockSpec((1, 1, D), lambda t, f, e: (e[0], 0, 0)),
        ],
        out_specs=pl.BlockSpec((BT, D), lambda t, f, e: (t, 0)),
    )
    out = pl.pallas_call(
        _ffn_body,
        grid_spec=grid_spec,
        out_shape=jax.ShapeDtypeStruct((tokens, D), jnp.float32),
        compiler_params=pltpu.CompilerParams(
            dimension_semantics=("parallel", "arbitrary"),
        ),
    )(e_idx, x2, W1, b1r, W2, b2r)
    return out.reshape(B, S, D)
